# baseline (speedup 1.0000x reference)
import functools

import jax
import jax.numpy as jnp
from jax import lax
from jax.experimental import pallas as pl
from jax.experimental.pallas import tpu as pltpu
from jax.experimental.pallas import tpu_sc as plsc

_SIZE = 32000
_N = 4096
_SMOOTHING = 0.1
_CONF = 1.0 - _SMOOTHING
_S = _SMOOTHING / (_SIZE - 2)

_BR = 16

_NC = 2
_NS = 16
_NW = _NC * _NS
_BPW = _N // _NW
_L = 16


def _sc_gather_body(x1_hbm, tgt_hbm, out_hbm, tgt_v, idx_v, xt_v, sem):
    wid = lax.axis_index("s") * _NC + lax.axis_index("c")
    base = wid * _BPW
    pltpu.sync_copy(tgt_hbm.at[pl.ds(base, _BPW)], tgt_v)
    for c in range(_BPW // _L):
        t16 = tgt_v[pl.ds(c * _L, _L)]
        i16 = base + c * _L + lax.iota(jnp.int32, _L)
        idx_v[pl.ds(c * _L, _L)] = i16 * _SIZE + t16
    pltpu.async_copy(x1_hbm.at[idx_v], xt_v, sem).wait()
    pltpu.sync_copy(xt_v, out_hbm.at[pl.ds(base, _BPW)])


_sc_gather = functools.partial(
    pl.kernel,
    mesh=plsc.VectorSubcoreMesh(core_axis_name="c", subcore_axis_name="s"),
    out_type=jax.ShapeDtypeStruct((_N,), jnp.float32),
    scratch_types=[
        pltpu.VMEM((_BPW,), jnp.int32),
        pltpu.VMEM((_BPW,), jnp.int32),
        pltpu.VMEM((_BPW,), jnp.float32),
        pltpu.SemaphoreType.DMA,
    ],
)(_sc_gather_body)


def _tc_body(x_ref, tgt_ref, xt_ref, out_ref, acc_ref):
    i = pl.program_id(0)

    @pl.when(i == 0)
    def _init():
        acc_ref[0] = 0.0

    xb = x_ref[...]
    m = jnp.max(xb, axis=1)
    se = jnp.sum(jnp.exp(xb - m[:, None]), axis=1)
    lse = m + jnp.log(se)
    sumx = jnp.sum(xb, axis=1)
    x0 = xb[:, 0]

    tgt = tgt_ref[0, 0, :]
    xt = xt_ref[0, 0, :]

    c0 = _CONF * jnp.log(_CONF) + (_SIZE - 2) * _S * jnp.log(_S)
    kl = jnp.where(tgt != 0, c0 + lse - _CONF * xt - _S * (sumx - x0 - xt), 0.0)
    acc_ref[0] += jnp.sum(kl)

    @pl.when(i == pl.num_programs(0) - 1)
    def _fin():
        out_ref[0] = acc_ref[0]


@jax.jit
def kernel(x, target):
    n, size = x.shape
    grid = n // _BR
    xt = _sc_gather(x.reshape(n * size), target)
    out = pl.pallas_call(
        _tc_body,
        grid=(grid,),
        in_specs=[
            pl.BlockSpec((_BR, size), lambda i: (i, 0)),
            pl.BlockSpec((1, 1, _BR), lambda i: (i, 0, 0)),
            pl.BlockSpec((1, 1, _BR), lambda i: (i, 0, 0)),
        ],
        out_specs=pl.BlockSpec(memory_space=pltpu.SMEM),
        out_shape=jax.ShapeDtypeStruct((1,), jnp.float32),
        scratch_shapes=[pltpu.SMEM((1,), jnp.float32)],
    )(x, target.reshape(grid, 1, _BR), xt.reshape(grid, 1, _BR))
    return out[0]


# speedup vs baseline: 2.5367x; 2.5367x over previous
import functools
import math

import jax
import jax.numpy as jnp
from jax import lax
from jax.experimental import pallas as pl
from jax.experimental.pallas import tpu as pltpu
from jax.experimental.pallas import tpu_sc as plsc

_SIZE = 32000
_N = 4096
_SMOOTHING = 0.1
_CONF = 1.0 - _SMOOTHING
_S = _SMOOTHING / (_SIZE - 2)
_C0 = _CONF * math.log(_CONF) + (_SIZE - 2) * _S * math.log(_S)

_BR = 32

_NC = 2
_NS = 16
_NW = _NC * _NS
_BPW = _N // _NW
_L = 16


def _tc_body(x_ref, tgt_ref, lse_ref, xt_ref, g_ref):
    xb = x_ref[...]
    m = jnp.max(xb, axis=1)
    se = jnp.sum(jnp.exp(xb - m[:, None]), axis=1)
    lse_ref[0, 0, :] = m + jnp.log(se)
    g_ref[0, 0, :] = jnp.sum(xb, axis=1) - xb[:, 0]

    tgt = tgt_ref[0, 0, :]
    col = lax.broadcasted_iota(jnp.int32, (_BR, _SIZE), 1)
    xt_ref[0, 0, :] = jnp.sum(jnp.where(col == tgt[:, None], xb, 0.0), axis=1)


def _sc_combine_body(lse_hbm, xt_hbm, g_hbm, tgt_hbm, out_hbm,
                     lse_v, xt_v, g_v, tgt_v, acc_v):
    wid = lax.axis_index("s") * _NC + lax.axis_index("c")
    base = wid * _BPW
    pltpu.sync_copy(lse_hbm.at[pl.ds(base, _BPW)], lse_v)
    pltpu.sync_copy(xt_hbm.at[pl.ds(base, _BPW)], xt_v)
    pltpu.sync_copy(g_hbm.at[pl.ds(base, _BPW)], g_v)
    pltpu.sync_copy(tgt_hbm.at[pl.ds(base, _BPW)], tgt_v)
    acc = jnp.zeros((_L,), jnp.float32)
    for c in range(_BPW // _L):
        sl = pl.ds(c * _L, _L)
        lse = lse_v[sl]
        xt = xt_v[sl]
        g = g_v[sl]
        tgt = tgt_v[sl]
        kl = _C0 + lse - _CONF * xt - _S * (g - xt)
        acc = acc + jnp.where(tgt != 0, kl, 0.0)
    acc_v[...] = acc
    pltpu.sync_copy(acc_v, out_hbm.at[wid])


_sc_combine = functools.partial(
    pl.kernel,
    mesh=plsc.VectorSubcoreMesh(core_axis_name="c", subcore_axis_name="s"),
    out_type=jax.ShapeDtypeStruct((_NW, _L), jnp.float32),
    scratch_types=[
        pltpu.VMEM((_BPW,), jnp.float32),
        pltpu.VMEM((_BPW,), jnp.float32),
        pltpu.VMEM((_BPW,), jnp.float32),
        pltpu.VMEM((_BPW,), jnp.int32),
        pltpu.VMEM((_L,), jnp.float32),
    ],
)(_sc_combine_body)


def _tc_final_body(p_ref, out_ref):
    out_ref[0] = jnp.sum(p_ref[...])


@jax.jit
def kernel(x, target):
    n, size = x.shape
    grid = n // _BR
    shp3 = jax.ShapeDtypeStruct((grid, 1, _BR), jnp.float32)
    blk3 = pl.BlockSpec((1, 1, _BR), lambda i: (i, 0, 0))
    lse3, xt3, g3 = pl.pallas_call(
        _tc_body,
        grid=(grid,),
        in_specs=[
            pl.BlockSpec((_BR, size), lambda i: (i, 0)),
            pl.BlockSpec((1, 1, _BR), lambda i: (i, 0, 0)),
        ],
        out_specs=[blk3, blk3, blk3],
        out_shape=[shp3, shp3, shp3],
    )(x, target.reshape(grid, 1, _BR))
    partials = _sc_combine(lse3.reshape(n), xt3.reshape(n), g3.reshape(n),
                           target)
    out = pl.pallas_call(
        _tc_final_body,
        out_specs=pl.BlockSpec(memory_space=pltpu.SMEM),
        out_shape=jax.ShapeDtypeStruct((1,), jnp.float32),
    )(partials)
    return out[0]


# speedup vs baseline: 3.0376x; 1.1975x over previous
import functools
import math

import jax
import jax.numpy as jnp
from jax import lax
from jax.experimental import pallas as pl
from jax.experimental.pallas import tpu as pltpu
from jax.experimental.pallas import tpu_sc as plsc

_SIZE = 32000
_N = 4096
_SMOOTHING = 0.1
_CONF = 1.0 - _SMOOTHING
_S = _SMOOTHING / (_SIZE - 2)
_C0 = _CONF * math.log(_CONF) + (_SIZE - 2) * _S * math.log(_S)

_BR = 64

_NC = 2
_NS = 16
_NW = _NC * _NS
_BPW = _N // _NW
_L = 16


def _tc_body(x_ref, tgt_ref, lse_ref, xt_ref, g_ref):
    xb = x_ref[...]
    m = jnp.max(xb, axis=1)
    se = jnp.sum(jnp.exp(xb - m[:, None]), axis=1)
    lse_ref[0, 0, :] = m + jnp.log(se)
    g_ref[0, 0, :] = jnp.sum(xb, axis=1) - xb[:, 0]

    tgt = tgt_ref[0, 0, :]
    col = lax.broadcasted_iota(jnp.int32, (_BR, _SIZE), 1)
    xt_ref[0, 0, :] = jnp.sum(jnp.where(col == tgt[:, None], xb, 0.0), axis=1)


def _sc_combine_body(lse_hbm, xt_hbm, g_hbm, tgt_hbm, out_hbm,
                     lse_v, xt_v, g_v, tgt_v, acc_v):
    wid = lax.axis_index("s") * _NC + lax.axis_index("c")
    base = wid * _BPW
    pltpu.sync_copy(lse_hbm.at[pl.ds(base, _BPW)], lse_v)
    pltpu.sync_copy(xt_hbm.at[pl.ds(base, _BPW)], xt_v)
    pltpu.sync_copy(g_hbm.at[pl.ds(base, _BPW)], g_v)
    pltpu.sync_copy(tgt_hbm.at[pl.ds(base, _BPW)], tgt_v)
    acc = jnp.zeros((_L,), jnp.float32)
    for c in range(_BPW // _L):
        sl = pl.ds(c * _L, _L)
        lse = lse_v[sl]
        xt = xt_v[sl]
        g = g_v[sl]
        tgt = tgt_v[sl]
        kl = _C0 + lse - _CONF * xt - _S * (g - xt)
        acc = acc + jnp.where(tgt != 0, kl, 0.0)
    acc_v[...] = acc
    pltpu.sync_copy(acc_v, out_hbm.at[wid])


_sc_combine = functools.partial(
    pl.kernel,
    mesh=plsc.VectorSubcoreMesh(core_axis_name="c", subcore_axis_name="s"),
    out_type=jax.ShapeDtypeStruct((_NW, _L), jnp.float32),
    scratch_types=[
        pltpu.VMEM((_BPW,), jnp.float32),
        pltpu.VMEM((_BPW,), jnp.float32),
        pltpu.VMEM((_BPW,), jnp.float32),
        pltpu.VMEM((_BPW,), jnp.int32),
        pltpu.VMEM((_L,), jnp.float32),
    ],
)(_sc_combine_body)


def _tc_final_body(p_ref, out_ref):
    out_ref[0] = jnp.sum(p_ref[...])


@jax.jit
def kernel(x, target):
    n, size = x.shape
    grid = n // _BR
    shp3 = jax.ShapeDtypeStruct((grid, 1, _BR), jnp.float32)
    blk3 = pl.BlockSpec((1, 1, _BR), lambda i: (i, 0, 0))
    lse3, xt3, g3 = pl.pallas_call(
        _tc_body,
        grid=(grid,),
        in_specs=[
            pl.BlockSpec((_BR, size), lambda i: (i, 0)),
            pl.BlockSpec((1, 1, _BR), lambda i: (i, 0, 0)),
        ],
        out_specs=[blk3, blk3, blk3],
        out_shape=[shp3, shp3, shp3],
    )(x, target.reshape(grid, 1, _BR))
    partials = _sc_combine(lse3.reshape(n), xt3.reshape(n), g3.reshape(n),
                           target)
    out = pl.pallas_call(
        _tc_final_body,
        out_specs=pl.BlockSpec(memory_space=pltpu.SMEM),
        out_shape=jax.ShapeDtypeStruct((1,), jnp.float32),
    )(partials)
    return out[0]


# speedup vs baseline: 3.3066x; 1.0885x over previous
import functools
import math

import jax
import jax.numpy as jnp
from jax import lax
from jax.experimental import pallas as pl
from jax.experimental.pallas import tpu as pltpu
from jax.experimental.pallas import tpu_sc as plsc

_SIZE = 32000
_N = 4096
_SMOOTHING = 0.1
_CONF = 1.0 - _SMOOTHING
_S = _SMOOTHING / (_SIZE - 2)
_C0 = _CONF * math.log(_CONF) + (_SIZE - 2) * _S * math.log(_S)

_BR = 64

_NC = 2
_NS = 16
_NW = _NC * _NS
_BPW = _N // _NW
_L = 16


def _tc_body(x_ref, tgt_ref, lse_ref, xt_ref, g_ref):
    xb = x_ref[...]
    m = jnp.max(xb, axis=1)
    se = jnp.sum(jnp.exp(xb - m[:, None]), axis=1)
    lse_ref[0, 0, :] = m + jnp.log(se)
    g_ref[0, 0, :] = jnp.sum(xb, axis=1) - xb[:, 0]

    lane_iota = lax.broadcasted_iota(jnp.int32, (1, 128), 1)
    for r in range(_BR):
        t = tgt_ref[0, 0, r]
        ch = pl.multiple_of((t // 128) * 128, 128)
        v = x_ref[pl.ds(r, 1), pl.ds(ch, 128)]
        xt_ref[0, 0, r] = jnp.sum(jnp.where(lane_iota == t - ch, v, 0.0))


def _sc_combine_body(lse_hbm, xt_hbm, g_hbm, tgt_hbm, out_hbm,
                     lse_v, xt_v, g_v, tgt_v, acc_v):
    wid = lax.axis_index("s") * _NC + lax.axis_index("c")
    base = wid * _BPW
    pltpu.sync_copy(lse_hbm.at[pl.ds(base, _BPW)], lse_v)
    pltpu.sync_copy(xt_hbm.at[pl.ds(base, _BPW)], xt_v)
    pltpu.sync_copy(g_hbm.at[pl.ds(base, _BPW)], g_v)
    pltpu.sync_copy(tgt_hbm.at[pl.ds(base, _BPW)], tgt_v)
    acc = jnp.zeros((_L,), jnp.float32)
    for c in range(_BPW // _L):
        sl = pl.ds(c * _L, _L)
        lse = lse_v[sl]
        xt = xt_v[sl]
        g = g_v[sl]
        tgt = tgt_v[sl]
        kl = _C0 + lse - _CONF * xt - _S * (g - xt)
        acc = acc + jnp.where(tgt != 0, kl, 0.0)
    acc_v[...] = acc
    pltpu.sync_copy(acc_v, out_hbm.at[wid])


_sc_combine = functools.partial(
    pl.kernel,
    mesh=plsc.VectorSubcoreMesh(core_axis_name="c", subcore_axis_name="s"),
    out_type=jax.ShapeDtypeStruct((_NW, _L), jnp.float32),
    scratch_types=[
        pltpu.VMEM((_BPW,), jnp.float32),
        pltpu.VMEM((_BPW,), jnp.float32),
        pltpu.VMEM((_BPW,), jnp.float32),
        pltpu.VMEM((_BPW,), jnp.int32),
        pltpu.VMEM((_L,), jnp.float32),
    ],
)(_sc_combine_body)


def _tc_final_body(p_ref, out_ref):
    out_ref[0] = jnp.sum(p_ref[...])


@jax.jit
def kernel(x, target):
    n, size = x.shape
    grid = n // _BR
    shp3 = jax.ShapeDtypeStruct((grid, 1, _BR), jnp.float32)
    blk3 = pl.BlockSpec((1, 1, _BR), lambda i: (i, 0, 0))
    lse3, xt3, g3 = pl.pallas_call(
        _tc_body,
        grid=(grid,),
        in_specs=[
            pl.BlockSpec((_BR, size), lambda i: (i, 0)),
            pl.BlockSpec((1, 1, _BR), lambda i: (i, 0, 0),
                         memory_space=pltpu.SMEM),
        ],
        out_specs=[blk3,
                   pl.BlockSpec((1, 1, _BR), lambda i: (i, 0, 0),
                                memory_space=pltpu.SMEM),
                   blk3],
        out_shape=[shp3, shp3, shp3],
    )(x, target.reshape(grid, 1, _BR))
    partials = _sc_combine(lse3.reshape(n), xt3.reshape(n), g3.reshape(n),
                           target)
    out = pl.pallas_call(
        _tc_final_body,
        out_specs=pl.BlockSpec(memory_space=pltpu.SMEM),
        out_shape=jax.ShapeDtypeStruct((1,), jnp.float32),
    )(partials)
    return out[0]
